# baseline (speedup 1.0000x reference)
import functools

import jax
import jax.numpy as jnp
from jax import lax
from jax.experimental import pallas as pl
from jax.experimental.pallas import tpu as pltpu
from jax.experimental.pallas import tpu_sc as plsc

_N = 1024 * 50 * 512
_NW = 32
_PER_W = _N // _NW
_CHUNK = 51200
_NCHUNK = _PER_W // _CHUNK


def _build_sc_copy():
    mesh = plsc.VectorSubcoreMesh(core_axis_name="c", subcore_axis_name="s")

    @functools.partial(
        pl.kernel,
        mesh=mesh,
        out_type=jax.ShapeDtypeStruct((_N,), jnp.float32),
        scratch_types=[
            pltpu.VMEM((_CHUNK,), jnp.float32),
            pltpu.VMEM((_CHUNK,), jnp.float32),
            pltpu.SemaphoreType.DMA,
            pltpu.SemaphoreType.DMA,
            pltpu.SemaphoreType.DMA,
            pltpu.SemaphoreType.DMA,
        ],
    )
    def sc_copy(x_hbm, o_hbm, buf0, buf1, isem0, isem1, osem0, osem1):
        wid = lax.axis_index("s") * 2 + lax.axis_index("c")
        base = wid * _PER_W
        bufs = (buf0, buf1)
        isems = (isem0, isem1)
        osems = (osem0, osem1)

        def in_copy(i):
            s = i % 2
            return pltpu.make_async_copy(
                x_hbm.at[pl.ds(base + i * _CHUNK, _CHUNK)], bufs[s], isems[s])

        def out_copy(i):
            s = i % 2
            return pltpu.make_async_copy(
                bufs[s], o_hbm.at[pl.ds(base + i * _CHUNK, _CHUNK)], osems[s])

        in_copy(0).start()
        if _NCHUNK > 1:
            in_copy(1).start()
        for i in range(_NCHUNK):
            in_copy(i).wait()
            out_copy(i).start()
            if i + 2 < _NCHUNK:
                out_copy(i).wait()
                in_copy(i + 2).start()
        for i in range(max(0, _NCHUNK - 2), _NCHUNK):
            out_copy(i).wait()

    return sc_copy


_sc_copy = _build_sc_copy()


def kernel(x, hippocampus, neocortex):
    B, S, H = x.shape
    out = _sc_copy(x.reshape(-1))
    return out.reshape(B, S, H)


# speedup vs baseline: 1.4904x; 1.4904x over previous
import functools

import jax
import jax.numpy as jnp
from jax import lax
from jax.experimental import pallas as pl
from jax.experimental.pallas import tpu as pltpu
from jax.experimental.pallas import tpu_sc as plsc

_B, _S, _H = 1024, 50, 512
_NW = 32
_ROWS_PER_W = _B // _NW
_CHUNK_ROWS = 2
_NCHUNK = _ROWS_PER_W // _CHUNK_ROWS


def _build_sc_copy():
    mesh = plsc.VectorSubcoreMesh(core_axis_name="c", subcore_axis_name="s")

    @functools.partial(
        pl.kernel,
        mesh=mesh,
        out_type=jax.ShapeDtypeStruct((_B, _S, _H), jnp.float32),
        scratch_types=[
            pltpu.VMEM((_CHUNK_ROWS, _S, _H), jnp.float32),
            pltpu.VMEM((_CHUNK_ROWS, _S, _H), jnp.float32),
            pltpu.SemaphoreType.DMA,
            pltpu.SemaphoreType.DMA,
            pltpu.SemaphoreType.DMA,
            pltpu.SemaphoreType.DMA,
        ],
        compiler_params=pltpu.CompilerParams(use_tc_tiling_on_sc=True),
    )
    def sc_copy(x_hbm, o_hbm, buf0, buf1, isem0, isem1, osem0, osem1):
        wid = lax.axis_index("s") * 2 + lax.axis_index("c")
        base = wid * _ROWS_PER_W
        bufs = (buf0, buf1)
        isems = (isem0, isem1)
        osems = (osem0, osem1)

        def in_copy(i):
            s = i % 2
            return pltpu.make_async_copy(
                x_hbm.at[pl.ds(base + i * _CHUNK_ROWS, _CHUNK_ROWS)],
                bufs[s], isems[s])

        def out_copy(i):
            s = i % 2
            return pltpu.make_async_copy(
                bufs[s],
                o_hbm.at[pl.ds(base + i * _CHUNK_ROWS, _CHUNK_ROWS)],
                osems[s])

        in_copy(0).start()
        if _NCHUNK > 1:
            in_copy(1).start()
        for i in range(_NCHUNK):
            in_copy(i).wait()
            out_copy(i).start()
            if i + 2 < _NCHUNK:
                out_copy(i).wait()
                in_copy(i + 2).start()
        for i in range(max(0, _NCHUNK - 2), _NCHUNK):
            out_copy(i).wait()

    return sc_copy


_sc_copy = _build_sc_copy()


def kernel(x, hippocampus, neocortex):
    return _sc_copy(x)
